# baseline (speedup 1.0000x reference)
import numpy as np
import jax
import jax.numpy as jnp
from jax import lax
from jax.experimental import pallas as pl
from jax.experimental.pallas import tpu as pltpu
from jax.experimental.pallas import tpu_sc as plsc

_N_LEVELS = 16
_BASE_RES = 16
_MAX_RES = 2048
_T = 2 ** 19
_F = 2
_N_POINTS = 262144
_growth = np.exp((np.log(_MAX_RES) - np.log(_BASE_RES)) / (_N_LEVELS - 1))
_RES = [int(np.floor(_BASE_RES * _growth ** l)) for l in range(_N_LEVELS)]
_P1 = np.uint32(2654435761).astype(np.int32)
_P2 = np.int32(805459861)
_MASK = np.int32(_T - 1)

_NC, _NS = 2, 16
_NW = _NC * _NS
_PER_W = _N_POINTS // _NW
_C = 1024
_NCHUNK = _PER_W // _C
_L = 16


def _body(x0_hbm, x1_hbm, x2_hbm, tbl_hbm, out_hbm, xv, frv, idxv, rowsv,
          outv, sem_g, sem_o):
    x_hbm = [x0_hbm, x1_hbm, x2_hbm]
    wid = lax.axis_index("s") * _NC + lax.axis_index("c")

    def chunk_body(chunk, carry):
        base = wid * _PER_W + chunk * _C
        for d in range(3):
            pltpu.sync_copy(x_hbm[d].at[pl.ds(base, _C)], xv[d])

        out_copies = []
        for l in range(_N_LEVELS):
            res = _RES[l]
            dense = (res + 1) ** 3 <= _T
            resf = jnp.float32(res)
            resi = jnp.int32(res)
            ofs2 = jnp.int32(2 * l * _T)

            def a_body(i, c, res=res, dense=dense, resf=resf, resi=resi,
                       ofs2=ofs2):
                s = pl.ds(i * _L, _L)
                lo, hi = [], []
                for d in range(3):
                    p = xv[d][s] * resf
                    ii = p.astype(jnp.int32)
                    frv[d][s] = p - ii.astype(jnp.float32)
                    lo.append(ii)
                    hi.append(jnp.minimum(ii + 1, resi))
                if dense:
                    r1 = jnp.int32(res + 1)
                    r2 = jnp.int32((res + 1) * (res + 1))
                    t1 = [lo[1] * r1, hi[1] * r1]
                    t2 = [lo[2] * r2, hi[2] * r2]
                else:
                    t1 = [lo[1] * _P1, hi[1] * _P1]
                    t2 = [lo[2] * _P2, hi[2] * _P2]
                for corner in range(8):
                    b0 = corner & 1
                    b1 = (corner >> 1) & 1
                    b2 = (corner >> 2) & 1
                    if dense:
                        row = [lo[0], hi[0]][b0] + t1[b1] + t2[b2]
                    else:
                        row = ([lo[0], hi[0]][b0] ^ t1[b1] ^ t2[b2]) & _MASK
                    e0 = row + row + ofs2
                    idxv[2 * corner][s] = e0
                    idxv[2 * corner + 1][s] = e0 + 1
                return c

            lax.fori_loop(0, _C // _L, a_body, 0)

            gathers = [pltpu.async_copy(tbl_hbm.at[idxv[j]], rowsv[j], sem_g)
                       for j in range(16)]
            for g in gathers:
                g.wait()

            def b_body(i, c, l=l):
                s = pl.ds(i * _L, _L)
                fr = [frv[d][s] for d in range(3)]
                om = [1.0 - f for f in fr]
                acc0 = jnp.zeros((_L,), jnp.float32)
                acc1 = jnp.zeros((_L,), jnp.float32)
                for corner in range(8):
                    b0 = corner & 1
                    b1 = (corner >> 1) & 1
                    b2 = (corner >> 2) & 1
                    w = ([om[0], fr[0]][b0] * [om[1], fr[1]][b1]) \
                        * [om[2], fr[2]][b2]
                    acc0 = acc0 + w * rowsv[2 * corner][s]
                    acc1 = acc1 + w * rowsv[2 * corner + 1][s]
                outv[2 * l][s] = acc0
                outv[2 * l + 1][s] = acc1
                return c

            lax.fori_loop(0, _C // _L, b_body, 0)

            for f in range(2):
                p = 2 * l + f
                out_copies.append(pltpu.async_copy(
                    outv[p], out_hbm.at[pl.ds(p * _N_POINTS + base, _C)],
                    sem_o))

        for oc in out_copies:
            oc.wait()
        return carry

    lax.fori_loop(0, _NCHUNK, chunk_body, 0)


def kernel(in_tensor, table):
    x0 = in_tensor[:, 0]
    x1 = in_tensor[:, 1]
    x2 = in_tensor[:, 2]
    tbl = table.reshape(_N_LEVELS * _T * _F)
    mesh = plsc.VectorSubcoreMesh(core_axis_name="c", subcore_axis_name="s")
    f = pl.kernel(
        _body,
        out_type=jax.ShapeDtypeStruct((2 * _N_LEVELS * _N_POINTS,),
                                      jnp.float32),
        mesh=mesh,
        scratch_types=[
            [pltpu.VMEM((_C,), jnp.float32) for _ in range(3)],
            [pltpu.VMEM((_C,), jnp.float32) for _ in range(3)],
            [pltpu.VMEM((_C,), jnp.int32) for _ in range(16)],
            [pltpu.VMEM((_C,), jnp.float32) for _ in range(16)],
            [pltpu.VMEM((_C,), jnp.float32) for _ in range(32)],
            pltpu.SemaphoreType.DMA,
            pltpu.SemaphoreType.DMA,
        ],
    )
    out = f(x0, x1, x2, tbl)
    return out.reshape(2 * _N_LEVELS, _N_POINTS).T
